# baseline (speedup 1.0000x reference)
import functools

import jax
import jax.numpy as jnp
from jax import lax
from jax.experimental import pallas as pl
from jax.experimental.pallas import tpu as pltpu
from jax.experimental.pallas import tpu_sc as plsc

N = 10000
F = 128
E = 320000
NC = 2
NS = 16
NW = NC * NS
EPW = E // NW
CK = 80
NCH = EPW // CK
G = 8
NG = 16
NCHP = NG * G
EPWP = NCHP * CK
EPAD = NW * EPWP
NB = 4
RPT = 632
NPAD = NS * RPT
DPT = 640
DPAD = NS * DPT

_mesh = plsc.VectorSubcoreMesh(core_axis_name="c", subcore_axis_name="s")


@functools.partial(
    pl.kernel,
    out_type=jax.ShapeDtypeStruct((NC, DPAD), jnp.float32),
    mesh=_mesh,
    scratch_types=[
        pltpu.VMEM((NCHP, CK), jnp.int32),
        pltpu.VMEM((CK,), jnp.float32),
        pltpu.VMEM_SHARED((DPAD,), jnp.float32),
        pltpu.SemaphoreType.DMA,
    ],
)
def _deg_kernel(dst_hbm, zeros_hbm, out_hbm, dst_v, ones_v, acc, dsem):
    c = lax.axis_index("c")
    s = lax.axis_index("s")
    w = c * NS + s
    pltpu.sync_copy(dst_hbm.at[w], dst_v)
    for k in range(CK // 16):
        ones_v[pl.ds(k * 16, 16)] = jnp.full((16,), 1.0, jnp.float32)
    pltpu.sync_copy(zeros_hbm, acc.at[pl.ds(s * DPT, DPT)])
    plsc.subcore_barrier()

    def body(j, carry):
        pltpu.async_copy(ones_v, acc.at[dst_v.at[j]], dsem, add=True)
        return carry

    lax.fori_loop(0, NCHP, body, 0)

    def dbody(j, carry):
        pltpu.make_async_copy(ones_v, acc.at[dst_v.at[0]], dsem).wait()
        return carry

    lax.fori_loop(0, NCHP, dbody, 0)
    plsc.subcore_barrier()
    pltpu.sync_copy(acc.at[pl.ds(s * DPT, DPT)], out_hbm.at[c, pl.ds(s * DPT, DPT)])


@functools.partial(
    pl.kernel,
    out_type=jax.ShapeDtypeStruct((NC, NPAD, F), jnp.float32),
    mesh=_mesh,
    scratch_types=[
        pltpu.VMEM((2, G, CK), jnp.int32),
        pltpu.VMEM((2, G, CK), jnp.int32),
        pltpu.VMEM((CK, F), jnp.float32),
        pltpu.VMEM((CK, F), jnp.float32),
        pltpu.VMEM((CK, F), jnp.float32),
        pltpu.VMEM((CK, F), jnp.float32),
        pltpu.VMEM_SHARED((NPAD, F), jnp.float32),
        [pltpu.SemaphoreType.DMA] * NB,
        [pltpu.SemaphoreType.DMA] * NB,
        pltpu.SemaphoreType.DMA,
        pltpu.SemaphoreType.DMA,
    ],
)
def _agg_kernel(g_hbm, src_hbm, dst_hbm, zeros_hbm, out_hbm,
                srcst, dstst, rows0, rows1, rows2, rows3, acc,
                gsems, ssems, issem, idsem):
    c = lax.axis_index("c")
    s = lax.axis_index("s")
    w = c * NS + s
    rows = [rows0, rows1, rows2, rows3]

    pltpu.sync_copy(src_hbm.at[w, pl.ds(0, G)], srcst.at[0])
    pltpu.sync_copy(dst_hbm.at[w, pl.ds(0, G)], dstst.at[0])
    pltpu.sync_copy(zeros_hbm, acc.at[pl.ds(s * RPT, RPT)])

    def stage(gn, pn):
        off = pl.multiple_of(gn * G, G)
        pltpu.async_copy(src_hbm.at[w, pl.ds(off, G)], srcst.at[pn], issem)
        pltpu.async_copy(dst_hbm.at[w, pl.ds(off, G)], dstst.at[pn], idsem)

    def gather(idx_row, b):
        pltpu.async_copy(g_hbm.at[idx_row], rows[b], gsems[b])

    def gather_wait(pp, b):
        pltpu.make_async_copy(g_hbm.at[srcst.at[pp, 0]], rows[b],
                              gsems[b]).wait()

    def scat(pp, k, b):
        pltpu.async_copy(rows[b], acc.at[dstst.at[pp, k]], ssems[b],
                         priority=1, add=True)

    def scat_wait(pp, b):
        pltpu.make_async_copy(rows[b], acc.at[dstst.at[pp, 0]],
                              ssems[b]).wait()

    def group(gn, pp, first, last):
        if not last:
            stage(gn + 1, 1 - pp)
        for k in range(G):
            b = k % NB
            b2 = (k + 2) % NB
            if not (first and k < 2):
                scat_wait(pp, b2)
            if k < G - 2:
                gather(srcst.at[pp, k + 2], b2)
            elif not last:
                if k == G - 2:
                    pltpu.make_async_copy(src_hbm.at[w, pl.ds(0, G)],
                                          srcst.at[1 - pp], issem).wait()
                    pltpu.make_async_copy(dst_hbm.at[w, pl.ds(0, G)],
                                          dstst.at[1 - pp], idsem).wait()
                gather(srcst.at[1 - pp, k - (G - 2)], b2)
            gather_wait(pp, b)
            scat(pp, k, b)

    gather(srcst.at[0, 0], 0)
    gather(srcst.at[0, 1], 1)
    plsc.subcore_barrier()

    group(0, 0, True, False)

    def body(gn, carry):
        group(gn, gn % 2, False, False)
        return carry

    lax.fori_loop(1, NG - 1, body, 0)
    group(NG - 1, (NG - 1) % 2, False, True)

    scat_wait((NG - 1) % 2, (NCHP - 2) % NB)
    scat_wait((NG - 1) % 2, (NCHP - 1) % NB)
    plsc.subcore_barrier()
    pltpu.sync_copy(acc.at[pl.ds(s * RPT, RPT)],
                    out_hbm.at[c, pl.ds(s * RPT, RPT)])


_BM = 2000


def _dinv_of(deg_blk):
    d = deg_blk[:, 0:1] + deg_blk[:, 1:2] + 1.0
    return lax.rsqrt(d)


def _tc_first(x, W1, degT):
    k = x.shape[1]

    def kern(x_ref, w_ref, deg_ref, out_ref):
        dinv = _dinv_of(deg_ref[...])
        out_ref[...] = dinv * jnp.dot(x_ref[...], w_ref[...],
                                      preferred_element_type=jnp.float32)

    return pl.pallas_call(
        kern,
        grid=(N // _BM,),
        in_specs=[
            pl.BlockSpec((_BM, k), lambda i: (i, 0)),
            pl.BlockSpec((k, F), lambda i: (0, 0)),
            pl.BlockSpec((_BM, 2), lambda i: (i, 0)),
        ],
        out_specs=pl.BlockSpec((_BM, F), lambda i: (i, 0)),
        out_shape=jax.ShapeDtypeStruct((N, F), jnp.float32),
    )(x, W1, degT)


def _tc_mid(g, a0, a1, degT, W, b2d):
    def kern(g_ref, a0_ref, a1_ref, deg_ref, w_ref, b_ref, out_ref):
        dinv = _dinv_of(deg_ref[...])
        h = jnp.maximum(
            dinv * (g_ref[...] + a0_ref[...] + a1_ref[...]) + b_ref[...], 0.0)
        out_ref[...] = dinv * jnp.dot(h, w_ref[...],
                                      preferred_element_type=jnp.float32)

    return pl.pallas_call(
        kern,
        grid=(N // N,),
        in_specs=[
            pl.BlockSpec((N, F), lambda i: (i, 0)),
            pl.BlockSpec((N, F), lambda i: (i, 0)),
            pl.BlockSpec((N, F), lambda i: (i, 0)),
            pl.BlockSpec((N, 2), lambda i: (i, 0)),
            pl.BlockSpec((F, F), lambda i: (0, 0)),
            pl.BlockSpec((1, F), lambda i: (0, 0)),
        ],
        out_specs=pl.BlockSpec((N, F), lambda i: (i, 0)),
        out_shape=jax.ShapeDtypeStruct((N, F), jnp.float32),
    )(g, a0, a1, degT, W, b2d)


def _tc_final(g, a0, a1, degT, b2d, fws, fbs):
    dims = [w.shape for w in fws]

    def kern(g_ref, a0_ref, a1_ref, deg_ref, b_ref,
             w1, w2, w3, w4, w5, w6, c1, c2, c3, c4, c5, c6, out_ref):
        dinv = _dinv_of(deg_ref[...])
        h = jnp.maximum(
            dinv * (g_ref[...] + a0_ref[...] + a1_ref[...]) + b_ref[...], 0.0)
        for w, c in ((w1, c1), (w2, c2), (w3, c3), (w4, c4), (w5, c5)):
            h = jnp.maximum(jnp.dot(h, w[...],
                                    preferred_element_type=jnp.float32)
                            + c[...], 0.0)
        out_ref[...] = jnp.dot(h, w6[...],
                               preferred_element_type=jnp.float32) + c6[...]

    in_specs = [
        pl.BlockSpec((N, F), lambda i: (i, 0)),
        pl.BlockSpec((N, F), lambda i: (i, 0)),
        pl.BlockSpec((N, F), lambda i: (i, 0)),
        pl.BlockSpec((N, 2), lambda i: (i, 0)),
        pl.BlockSpec((1, F), lambda i: (0, 0)),
    ]
    in_specs += [pl.BlockSpec(d, lambda i: (0, 0)) for d in dims]
    in_specs += [pl.BlockSpec((1, w.shape[1]), lambda i: (0, 0)) for w in fws]
    return pl.pallas_call(
        kern,
        grid=(N // N,),
        in_specs=in_specs,
        out_specs=pl.BlockSpec((N, dims[-1][1]), lambda i: (i, 0)),
        out_shape=jax.ShapeDtypeStruct((N, dims[-1][1]), jnp.float32),
    )(g, a0, a1, degT, b2d, *fws, *[b.reshape(1, -1) for b in fbs])


def kernel(x, edge_index, W1, b1, W2, b2, fw1, fb1, fw2, fb2, fw3, fb3,
           fw4, fb4, fw5, fb5, fw6, fb6):
    pad_i = jnp.arange(EPAD - E, dtype=jnp.int32)
    src3 = jnp.concatenate(
        [edge_index[0], pad_i % N]).reshape(NW, NCHP, CK)
    dst3 = jnp.concatenate(
        [edge_index[1], N + pad_i % (NPAD - N)]).reshape(NW, NCHP, CK)
    zeros1 = jnp.zeros((DPT,), jnp.float32)
    zeros2 = jnp.zeros((RPT, F), jnp.float32)

    deg_out = _deg_kernel(dst3, zeros1)
    degT = deg_out[:, :N].T

    b1r = b1.reshape(1, F)
    b2r = b2.reshape(1, F)
    fws = [fw1, fw2, fw3, fw4, fw5, fw6]
    fbs = [fb1, fb2, fb3, fb4, fb5, fb6]

    g = _tc_first(x, W1, degT)
    for l in range(10):
        aggf = _agg_kernel(g, src3, dst3, zeros2)
        a0 = aggf[0, :N]
        a1 = aggf[1, :N]
        bl = b1r if l == 0 else b2r
        if l < 9:
            g = _tc_mid(g, a0, a1, degT, W2, bl)
        else:
            out = _tc_final(g, a0, a1, degT, bl, fws, fbs)
    return out


# speedup vs baseline: 1.0070x; 1.0070x over previous
import functools

import jax
import jax.numpy as jnp
from jax import lax
from jax.experimental import pallas as pl
from jax.experimental.pallas import tpu as pltpu
from jax.experimental.pallas import tpu_sc as plsc

N = 10000
F = 128
E = 320000
NC = 2
NS = 16
NW = NC * NS
EPW = E // NW
CK = 80
NCH = EPW // CK
G = 8
NG = 16
NCHP = NG * G
EPWP = NCHP * CK
EPAD = NW * EPWP
NB = 4
RPT = 632
NPAD = NS * RPT
DPT = 640
DPAD = NS * DPT

_mesh = plsc.VectorSubcoreMesh(core_axis_name="c", subcore_axis_name="s")


@functools.partial(
    pl.kernel,
    out_type=jax.ShapeDtypeStruct((NC, DPAD), jnp.float32),
    mesh=_mesh,
    scratch_types=[
        pltpu.VMEM((NCHP, CK), jnp.int32),
        pltpu.VMEM((CK,), jnp.float32),
        pltpu.VMEM_SHARED((DPAD,), jnp.float32),
        pltpu.SemaphoreType.DMA,
    ],
)
def _deg_kernel(dst_hbm, zeros_hbm, out_hbm, dst_v, ones_v, acc, dsem):
    c = lax.axis_index("c")
    s = lax.axis_index("s")
    w = c * NS + s
    pltpu.sync_copy(dst_hbm.at[w], dst_v)
    for k in range(CK // 16):
        ones_v[pl.ds(k * 16, 16)] = jnp.full((16,), 1.0, jnp.float32)
    pltpu.sync_copy(zeros_hbm, acc.at[pl.ds(s * DPT, DPT)])
    plsc.subcore_barrier()

    def body(j, carry):
        pltpu.async_copy(ones_v, acc.at[dst_v.at[j]], dsem, add=True)
        return carry

    lax.fori_loop(0, NCHP, body, 0)

    def dbody(j, carry):
        pltpu.make_async_copy(ones_v, acc.at[dst_v.at[0]], dsem).wait()
        return carry

    lax.fori_loop(0, NCHP, dbody, 0)
    plsc.subcore_barrier()
    pltpu.sync_copy(acc.at[pl.ds(s * DPT, DPT)], out_hbm.at[c, pl.ds(s * DPT, DPT)])


@functools.partial(
    pl.kernel,
    out_type=jax.ShapeDtypeStruct((NC, NPAD, F), jnp.float32),
    mesh=_mesh,
    scratch_types=[
        pltpu.VMEM((2, G, CK), jnp.int32),
        pltpu.VMEM((2, G, CK), jnp.int32),
        pltpu.VMEM((CK, F), jnp.float32),
        pltpu.VMEM((CK, F), jnp.float32),
        pltpu.VMEM((CK, F), jnp.float32),
        pltpu.VMEM((CK, F), jnp.float32),
        pltpu.VMEM_SHARED((NPAD, F), jnp.float32),
        [pltpu.SemaphoreType.DMA] * NB,
        [pltpu.SemaphoreType.DMA] * NB,
        pltpu.SemaphoreType.DMA,
        pltpu.SemaphoreType.DMA,
    ],
)
def _agg_kernel(g_hbm, src_hbm, dst_hbm, zeros_hbm, out_hbm,
                srcst, dstst, rows0, rows1, rows2, rows3, acc,
                gsems, ssems, issem, idsem):
    c = lax.axis_index("c")
    s = lax.axis_index("s")
    w = c * NS + s
    rows = [rows0, rows1, rows2, rows3]

    pltpu.sync_copy(src_hbm.at[w, pl.ds(0, G)], srcst.at[0])
    pltpu.sync_copy(dst_hbm.at[w, pl.ds(0, G)], dstst.at[0])
    pltpu.sync_copy(zeros_hbm, acc.at[pl.ds(s * RPT, RPT)])

    def stage(gn, pn):
        off = pl.multiple_of(gn * G, G)
        pltpu.async_copy(src_hbm.at[w, pl.ds(off, G)], srcst.at[pn], issem)
        pltpu.async_copy(dst_hbm.at[w, pl.ds(off, G)], dstst.at[pn], idsem)

    def gather(idx_row, b):
        pltpu.async_copy(g_hbm.at[idx_row], rows[b], gsems[b])

    def gather_wait(pp, b):
        pltpu.make_async_copy(g_hbm.at[srcst.at[pp, 0]], rows[b],
                              gsems[b]).wait()

    def scat(pp, k, b):
        pltpu.async_copy(rows[b], acc.at[dstst.at[pp, k]], ssems[b],
                         priority=1, add=True)

    def scat_wait(pp, b):
        pltpu.make_async_copy(rows[b], acc.at[dstst.at[pp, 0]],
                              ssems[b]).wait()

    def group(gn, pp, first, last):
        if not last:
            stage(gn + 1, 1 - pp)
        for k in range(G):
            b = k % NB
            b2 = (k + 2) % NB
            if not (first and k < 2):
                scat_wait(pp, b2)
            if k < G - 2:
                gather(srcst.at[pp, k + 2], b2)
            elif not last:
                if k == G - 2:
                    pltpu.make_async_copy(src_hbm.at[w, pl.ds(0, G)],
                                          srcst.at[1 - pp], issem).wait()
                    pltpu.make_async_copy(dst_hbm.at[w, pl.ds(0, G)],
                                          dstst.at[1 - pp], idsem).wait()
                gather(srcst.at[1 - pp, k - (G - 2)], b2)
            gather_wait(pp, b)
            scat(pp, k, b)

    gather(srcst.at[0, 0], 0)
    gather(srcst.at[0, 1], 1)
    plsc.subcore_barrier()

    group(0, 0, True, False)

    def body(gn, carry):
        group(gn, gn % 2, False, False)
        return carry

    lax.fori_loop(1, NG - 1, body, 0)
    group(NG - 1, (NG - 1) % 2, False, True)

    scat_wait((NG - 1) % 2, (NCHP - 2) % NB)
    scat_wait((NG - 1) % 2, (NCHP - 1) % NB)
    plsc.subcore_barrier()
    pltpu.sync_copy(acc.at[pl.ds(s * RPT, RPT)],
                    out_hbm.at[c, pl.ds(s * RPT, RPT)])


_BM = 2000


def _dinv_of(deg_blk):
    d = deg_blk[:, 0:1] + deg_blk[:, 1:2] + 1.0
    return lax.rsqrt(d)


def _tc_first(x, W1, degT):
    k = x.shape[1]

    def kern(x_ref, w_ref, deg_ref, out_ref):
        dinv = _dinv_of(deg_ref[...])
        out_ref[...] = dinv * jnp.dot(x_ref[...], w_ref[...],
                                      preferred_element_type=jnp.float32)

    return pl.pallas_call(
        kern,
        grid=(N // _BM,),
        in_specs=[
            pl.BlockSpec((_BM, k), lambda i: (i, 0)),
            pl.BlockSpec((k, F), lambda i: (0, 0)),
            pl.BlockSpec((_BM, 2), lambda i: (i, 0)),
        ],
        out_specs=pl.BlockSpec((_BM, F), lambda i: (i, 0)),
        out_shape=jax.ShapeDtypeStruct((N, F), jnp.float32),
    )(x, W1, degT)


def _tc_mid(g, a0, a1, degT, W, b2d):
    def kern(g_ref, a0_ref, a1_ref, deg_ref, w_ref, b_ref, out_ref):
        dinv = _dinv_of(deg_ref[...])
        h = jnp.maximum(
            dinv * (g_ref[...] + a0_ref[...] + a1_ref[...]) + b_ref[...], 0.0)
        out_ref[...] = dinv * jnp.dot(h, w_ref[...],
                                      preferred_element_type=jnp.float32)

    return pl.pallas_call(
        kern,
        grid=(N // _BM,),
        in_specs=[
            pl.BlockSpec((_BM, F), lambda i: (i, 0)),
            pl.BlockSpec((_BM, F), lambda i: (i, 0)),
            pl.BlockSpec((_BM, F), lambda i: (i, 0)),
            pl.BlockSpec((_BM, 2), lambda i: (i, 0)),
            pl.BlockSpec((F, F), lambda i: (0, 0)),
            pl.BlockSpec((1, F), lambda i: (0, 0)),
        ],
        out_specs=pl.BlockSpec((_BM, F), lambda i: (i, 0)),
        out_shape=jax.ShapeDtypeStruct((N, F), jnp.float32),
    )(g, a0, a1, degT, W, b2d)


def _tc_final(g, a0, a1, degT, b2d, fws, fbs):
    dims = [w.shape for w in fws]

    def kern(g_ref, a0_ref, a1_ref, deg_ref, b_ref,
             w1, w2, w3, w4, w5, w6, c1, c2, c3, c4, c5, c6, out_ref):
        dinv = _dinv_of(deg_ref[...])
        h = jnp.maximum(
            dinv * (g_ref[...] + a0_ref[...] + a1_ref[...]) + b_ref[...], 0.0)
        for w, c in ((w1, c1), (w2, c2), (w3, c3), (w4, c4), (w5, c5)):
            h = jnp.maximum(jnp.dot(h, w[...],
                                    preferred_element_type=jnp.float32)
                            + c[...], 0.0)
        out_ref[...] = jnp.dot(h, w6[...],
                               preferred_element_type=jnp.float32) + c6[...]

    in_specs = [
        pl.BlockSpec((_BM, F), lambda i: (i, 0)),
        pl.BlockSpec((_BM, F), lambda i: (i, 0)),
        pl.BlockSpec((_BM, F), lambda i: (i, 0)),
        pl.BlockSpec((_BM, 2), lambda i: (i, 0)),
        pl.BlockSpec((1, F), lambda i: (0, 0)),
    ]
    in_specs += [pl.BlockSpec(d, lambda i: (0, 0)) for d in dims]
    in_specs += [pl.BlockSpec((1, w.shape[1]), lambda i: (0, 0)) for w in fws]
    return pl.pallas_call(
        kern,
        grid=(N // _BM,),
        in_specs=in_specs,
        out_specs=pl.BlockSpec((_BM, dims[-1][1]), lambda i: (i, 0)),
        out_shape=jax.ShapeDtypeStruct((N, dims[-1][1]), jnp.float32),
    )(g, a0, a1, degT, b2d, *fws, *[b.reshape(1, -1) for b in fbs])


def kernel(x, edge_index, W1, b1, W2, b2, fw1, fb1, fw2, fb2, fw3, fb3,
           fw4, fb4, fw5, fb5, fw6, fb6):
    pad_i = jnp.arange(EPAD - E, dtype=jnp.int32)
    src3 = jnp.concatenate(
        [edge_index[0], pad_i % N]).reshape(NW, NCHP, CK)
    dst3 = jnp.concatenate(
        [edge_index[1], N + pad_i % (NPAD - N)]).reshape(NW, NCHP, CK)
    zeros1 = jnp.zeros((DPT,), jnp.float32)
    zeros2 = jnp.zeros((RPT, F), jnp.float32)

    deg_out = _deg_kernel(dst3, zeros1)
    degT = deg_out[:, :N].T

    b1r = b1.reshape(1, F)
    b2r = b2.reshape(1, F)
    fws = [fw1, fw2, fw3, fw4, fw5, fw6]
    fbs = [fb1, fb2, fb3, fb4, fb5, fb6]

    g = _tc_first(x, W1, degT)
    for l in range(10):
        aggf = _agg_kernel(g, src3, dst3, zeros2)
        a0 = aggf[0, :N]
        a1 = aggf[1, :N]
        bl = b1r if l == 0 else b2r
        if l < 9:
            g = _tc_mid(g, a0, a1, degT, W2, bl)
        else:
            out = _tc_final(g, a0, a1, degT, bl, fws, fbs)
    return out
